# baseline (speedup 1.0000x reference)
import functools

import jax
import jax.numpy as jnp
from jax import lax
from jax.experimental import pallas as pl
from jax.experimental.pallas import tpu as pltpu
from jax.experimental.pallas import tpu_sc as plsc

_NC = 2
_NS = 16
_NW = _NC * _NS
_LANES = 16

_GRP = 100


def _emb_body(seq, emb, rows_w, ch_rows, n_grp,
              ids, tok, pos, out, idx_v, rows_v, pos_v, sem):
    wid = lax.axis_index("s") * _NC + lax.axis_index("c")
    row0 = wid * rows_w
    g0 = row0 // _GRP
    n_ch = rows_w // ch_rows

    pltpu.sync_copy(pos, pos_v)

    def chunk(c, carry):
        pltpu.sync_copy(
            ids.at[pl.ds(pl.multiple_of(g0 + c * n_grp, 8), n_grp)], idx_v)
        cps = []
        for j in range(n_grp):
            cps.append(
                pltpu.async_copy(
                    tok.at[idx_v.at[j]],
                    rows_v.at[pl.ds(j * _GRP, _GRP)],
                    sem,
                )
            )
        for cp in cps:
            cp.wait()

        def addrow(r, carry2):
            p = lax.rem(r, seq)
            for d in range(emb // _LANES):
                sl = pl.ds(d * _LANES, _LANES)
                rows_v[r, sl] = rows_v[r, sl] + pos_v[p, sl]
            return carry2

        lax.fori_loop(0, ch_rows, addrow, 0, unroll=2)

        pltpu.sync_copy(
            rows_v, out.at[pl.ds(pl.multiple_of(row0 + c * ch_rows, 8),
                                 ch_rows)])
        return carry

    lax.fori_loop(0, n_ch, chunk, 0)


@functools.partial(jax.jit, static_argnames=("batch", "seq", "emb", "s_ch"))
def _emb_call(ids2d, token_table, pos_table, *, batch, seq, emb, s_ch):
    rows = batch * seq
    rows_w = rows // _NW
    ch_rows = s_ch * seq
    n_grp = ch_rows // _GRP

    mesh = plsc.VectorSubcoreMesh(core_axis_name="c", subcore_axis_name="s")
    kern = functools.partial(
        pl.kernel,
        out_type=jax.ShapeDtypeStruct((rows, emb), jnp.float32),
        mesh=mesh,
        scratch_types=[
            pltpu.VMEM((n_grp, _GRP), jnp.int32),
            pltpu.VMEM((ch_rows, emb), jnp.float32),
            pltpu.VMEM((seq, emb), jnp.float32),
            pltpu.SemaphoreType.DMA,
        ],
        compiler_params=pltpu.CompilerParams(use_tc_tiling_on_sc=False),
    )(functools.partial(_emb_body, seq, emb, rows_w, ch_rows, n_grp))
    return kern(ids2d, token_table, pos_table)


def kernel(input_ids, token_table, pos_table):
    batch, seq = input_ids.shape
    emb = token_table.shape[1]
    ids2d = input_ids.astype(jnp.int32).reshape(-1, _GRP)
    out = _emb_call(ids2d, token_table, pos_table,
                    batch=batch, seq=seq, emb=emb, s_ch=4)
    return out.reshape(batch, seq, emb)
